# baseline (speedup 1.0000x reference)
import jax
import jax.numpy as jnp
import numpy as np
from jax.experimental import pallas as pl
from jax.experimental.pallas import tpu as pltpu

_LOG_2PI = float(np.log(2.0 * np.pi))


def _prep_body(L_ref, mu_ref, w_ref, At_ref):
    K, D, _ = L_ref.shape
    C = K * D
    r2 = jax.lax.broadcasted_iota(jnp.int32, (D, D), 0)
    c2 = jax.lax.broadcasted_iota(jnp.int32, (D, D), 1)
    tril_m = r2 >= c2
    eye2 = (r2 == c2).astype(jnp.float32)
    sig_list = []
    for k in range(K):
        Lt = jnp.where(tril_m, L_ref[k], 0.0)
        Sig = jax.lax.dot_general(Lt, Lt, (((1,), (1,)), ((), ())),
                                  preferred_element_type=jnp.float32) + eye2
        sig_list.append(Sig)
    M2 = jnp.concatenate(sig_list, axis=1)

    lane = jax.lax.broadcasted_iota(jnp.int32, (1, C), 1)
    lane_mod = jnp.bitwise_and(lane, D - 1)
    row_i = jax.lax.broadcasted_iota(jnp.int32, (D, 1), 0)
    sel = (jax.lax.broadcasted_iota(jnp.int32, (K, C), 0) ==
           jax.lax.broadcasted_iota(jnp.int32, (K, C), 1) // D).astype(jnp.float32)
    edig = (jnp.bitwise_and(jax.lax.broadcasted_iota(jnp.int32, (C, D), 0), D - 1)
            == jax.lax.broadcasted_iota(jnp.int32, (C, D), 1)).astype(jnp.float32)

    def body(i, carry):
        M, pprod = carry
        rm = row_i == i
        cm = lane_mod == i
        r = jnp.sum(jnp.where(rm, M, 0.0), axis=0, keepdims=True)
        z = jnp.where(cm, r, 0.0)
        z = pltpu.roll(z, -i, 1)
        for sh in (1, 2, 4, 8, 16, 32):
            z = z + pltpu.roll(z, sh, 1)
        pinv = 1.0 / z
        rp = r * pinv
        r32 = jax.lax.dot_general(sel * r, edig, (((1,), (0,)), ((), ())),
                                  preferred_element_type=jnp.float32)
        craw = jax.lax.dot_general(r32, sel,
                                   (((0,), (0,)), ((), ())),
                                   preferred_element_type=jnp.float32)
        cs = craw * jnp.where(row_i < i, -1.0, 1.0)
        upd = M - cs * rp
        rowpatch = jnp.where(cm, pinv, rp)
        Mn = jnp.where(rm, rowpatch, jnp.where(cm, cs * (-pinv), upd))
        return Mn, pprod * z

    A2, pprod = jax.lax.fori_loop(
        0, D, body, (M2, jnp.ones((1, C), jnp.float32)))

    ld_row = jnp.log(pprod)
    selp = (jax.lax.broadcasted_iota(jnp.int32, (C, K), 0) ==
            jax.lax.broadcasted_iota(jnp.int32, (C, K), 1) * D).astype(jnp.float32)
    ld2 = jax.lax.dot_general(ld_row, selp, (((1,), (0,)), ((), ())),
                              preferred_element_type=jnp.float32)
    colmask = (jax.lax.broadcasted_iota(jnp.int32, (C, K), 0) // D ==
               jax.lax.broadcasted_iota(jnp.int32, (C, K), 1))
    muT = jax.lax.dot_general(eye2, mu_ref[...], (((1,), (1,)), ((), ())),
                              preferred_element_type=jnp.float32)
    mu_g = jax.lax.dot_general(edig, muT, (((1,), (0,)), ((), ())),
                               preferred_element_type=jnp.float32)
    musel = jnp.where(colmask, mu_g, 0.0)
    Bm = jax.lax.dot_general(A2, musel, (((1,), (0,)), ((), ())),
                             preferred_element_type=jnp.float32)
    BtMu = jax.lax.dot_general(Bm, mu_ref[...], (((0,), (1,)), ((), ())),
                               preferred_element_type=jnp.float32)
    kk1 = jax.lax.broadcasted_iota(jnp.int32, (K, K), 0)
    kk2 = jax.lax.broadcasted_iota(jnp.int32, (K, K), 1)
    cdiag = jnp.sum(jnp.where(kk1 == kk2, BtMu, 0.0), axis=0, keepdims=True)
    w = w_ref[...]
    wm = jnp.max(w)
    logw = w - (wm + jnp.log(jnp.sum(jnp.exp(w - wm))))
    beta = -0.5 * (D * _LOG_2PI + ld2) + logw - 0.5 * cdiag

    pad = D - K
    top = jnp.concatenate(
        [-0.5 * A2, eye2, Bm, jnp.zeros((D, pad), jnp.float32)], axis=1)
    botrow = jnp.concatenate(
        [jnp.zeros((1, C + D), jnp.float32), beta,
         jnp.zeros((1, pad), jnp.float32)], axis=1)
    bottom = jnp.concatenate(
        [botrow, jnp.zeros((D - 1, C + D + K + pad), jnp.float32)], axis=0)
    At_ref[...] = jnp.concatenate([top, bottom], axis=0)


def _maha_body(X_ref, At_ref, m_ref, s_ref):
    j = pl.program_id(1)
    nj = pl.num_programs(1)
    Xb = X_ref[...]
    BN, D = Xb.shape
    KD = At_ref.shape[1] - 2 * D
    K = KD // D
    ones_lane = (jax.lax.broadcasted_iota(jnp.int32, (BN, D), 1) == 0)
    Xaug = jnp.concatenate(
        [Xb, jnp.where(ones_lane, 1.0, 0.0)], axis=1)
    TtF = jax.lax.dot_general(At_ref[...], Xaug, (((0,), (1,)), ((), ())),
                              preferred_element_type=jnp.float32)
    Tq = TtF[0:KD].reshape(K, D, BN)
    Xt3 = TtF[KD:KD + D].reshape(1, D, BN)
    lb = TtF[KD + D:KD + D + K]
    qT = jnp.sum(Tq * Xt3, axis=1)
    logp = lb + qT
    mk = jnp.max(logp, axis=0, keepdims=True)
    ss = jnp.sum(jnp.exp(logp - mk), axis=0, keepdims=True)

    @pl.when(j == 0)
    def _():
        m_ref[...] = mk.reshape(1, 1, BN)
        s_ref[...] = ss.reshape(1, 1, BN)

    @pl.when(j > 0)
    def _():
        mp = m_ref[...].reshape(1, BN)
        sp = s_ref[...].reshape(1, BN)
        mn = jnp.maximum(mp, mk)
        s_ref[...] = (sp * jnp.exp(mp - mn) + ss * jnp.exp(mk - mn)).reshape(1, 1, BN)
        m_ref[...] = mn.reshape(1, 1, BN)

    @pl.when(j == nj - 1)
    def _():
        mv = m_ref[...].reshape(1, BN)
        sv = s_ref[...].reshape(1, BN)
        mtot = jnp.max(mv)
        stot = jnp.sum(sv * jnp.exp(mv - mtot))
        m_ref[...] = jnp.full((1, 1, BN), mtot, jnp.float32)
        s_ref[...] = jnp.full((1, 1, BN), stot, jnp.float32)


def kernel(X, mu, L, weights, it):
    N, D = X.shape
    K = mu.shape[0]
    CF = K * D + 2 * D
    w2 = weights.reshape(1, K)
    At = pl.pallas_call(
        _prep_body,
        out_shape=jax.ShapeDtypeStruct((2 * D, CF), jnp.float32),
    )(L, mu, w2)

    BN = 512
    PAR = 8
    SEQ = N // (PAR * BN)
    m, s = pl.pallas_call(
        _maha_body,
        grid=(PAR, SEQ),
        in_specs=[pl.BlockSpec((BN, D), lambda i, j: (i * SEQ + j, 0)),
                  pl.BlockSpec((2 * D, CF), lambda i, j: (0, 0))],
        out_specs=[pl.BlockSpec((1, 1, BN), lambda i, j: (i, 0, 0)),
                   pl.BlockSpec((1, 1, BN), lambda i, j: (i, 0, 0))],
        out_shape=[jax.ShapeDtypeStruct((PAR, 1, BN), jnp.float32),
                   jax.ShapeDtypeStruct((PAR, 1, BN), jnp.float32)],
        compiler_params=pltpu.CompilerParams(
            dimension_semantics=("parallel", "arbitrary")),
    )(X, At)

    mv = m[:, 0, 0]
    sv = s[:, 0, 0]
    Mx = jnp.max(mv)
    return -(Mx + jnp.log(jnp.sum(sv * jnp.exp(mv - Mx))))


# speedup vs baseline: 1.0829x; 1.0829x over previous
import jax
import jax.numpy as jnp
import numpy as np
from jax.experimental import pallas as pl
from jax.experimental.pallas import tpu as pltpu

_LOG_2PI = float(np.log(2.0 * np.pi))


def _prep_body(L_ref, mu_ref, w_ref, At_ref):
    K, D, _ = L_ref.shape
    C = K * D
    r2 = jax.lax.broadcasted_iota(jnp.int32, (D, D), 0)
    c2 = jax.lax.broadcasted_iota(jnp.int32, (D, D), 1)
    tril_m = r2 >= c2
    eye2 = (r2 == c2).astype(jnp.float32)
    sig_list = []
    for k in range(K):
        Lt = jnp.where(tril_m, L_ref[k], 0.0)
        Sig = jax.lax.dot_general(Lt, Lt, (((1,), (1,)), ((), ())),
                                  preferred_element_type=jnp.float32) + eye2
        sig_list.append(Sig)
    M2 = jnp.concatenate(sig_list, axis=1)

    lane = jax.lax.broadcasted_iota(jnp.int32, (1, C), 1)
    lane_mod = jnp.bitwise_and(lane, D - 1)
    row_i = jax.lax.broadcasted_iota(jnp.int32, (D, 1), 0)
    sel = (jax.lax.broadcasted_iota(jnp.int32, (K, C), 0) ==
           jax.lax.broadcasted_iota(jnp.int32, (K, C), 1) // D).astype(jnp.float32)
    edig = (jnp.bitwise_and(jax.lax.broadcasted_iota(jnp.int32, (C, D), 0), D - 1)
            == jax.lax.broadcasted_iota(jnp.int32, (C, D), 1)).astype(jnp.float32)

    def body(i, carry):
        M, pprod = carry
        rm = row_i == i
        cm = lane_mod == i
        r = jnp.sum(jnp.where(rm, M, 0.0), axis=0, keepdims=True)
        r32 = jax.lax.dot_general(sel * r, edig, (((1,), (0,)), ((), ())),
                                  preferred_element_type=jnp.float32)
        craw = jax.lax.dot_general(r32, sel,
                                   (((0,), (0,)), ((), ())),
                                   preferred_element_type=jnp.float32)
        p_bc = jnp.sum(jnp.where(rm, craw, 0.0), axis=0, keepdims=True)
        pinv = 1.0 / p_bc
        rp = r * pinv
        cs = craw * jnp.where(row_i < i, -1.0, 1.0)
        upd = M - cs * rp
        rowpatch = jnp.where(cm, pinv, rp)
        Mn = jnp.where(rm, rowpatch, jnp.where(cm, cs * (-pinv), upd))
        return Mn, pprod * p_bc

    A2, pprod = jax.lax.fori_loop(
        0, D, body, (M2, jnp.ones((1, C), jnp.float32)))

    ld_row = jnp.log(pprod)
    selp = (jax.lax.broadcasted_iota(jnp.int32, (C, K), 0) ==
            jax.lax.broadcasted_iota(jnp.int32, (C, K), 1) * D).astype(jnp.float32)
    ld2 = jax.lax.dot_general(ld_row, selp, (((1,), (0,)), ((), ())),
                              preferred_element_type=jnp.float32)
    colmask = (jax.lax.broadcasted_iota(jnp.int32, (C, K), 0) // D ==
               jax.lax.broadcasted_iota(jnp.int32, (C, K), 1))
    muT = jax.lax.dot_general(eye2, mu_ref[...], (((1,), (1,)), ((), ())),
                              preferred_element_type=jnp.float32)
    mu_g = jax.lax.dot_general(edig, muT, (((1,), (0,)), ((), ())),
                               preferred_element_type=jnp.float32)
    musel = jnp.where(colmask, mu_g, 0.0)
    Bm = jax.lax.dot_general(A2, musel, (((1,), (0,)), ((), ())),
                             preferred_element_type=jnp.float32)
    BtMu = jax.lax.dot_general(Bm, mu_ref[...], (((0,), (1,)), ((), ())),
                               preferred_element_type=jnp.float32)
    kk1 = jax.lax.broadcasted_iota(jnp.int32, (K, K), 0)
    kk2 = jax.lax.broadcasted_iota(jnp.int32, (K, K), 1)
    cdiag = jnp.sum(jnp.where(kk1 == kk2, BtMu, 0.0), axis=0, keepdims=True)
    w = w_ref[...]
    wm = jnp.max(w)
    logw = w - (wm + jnp.log(jnp.sum(jnp.exp(w - wm))))
    beta = -0.5 * (D * _LOG_2PI + ld2) + logw - 0.5 * cdiag

    pad = D - K
    top = jnp.concatenate(
        [-0.5 * A2, eye2, Bm, jnp.zeros((D, pad), jnp.float32)], axis=1)
    botrow = jnp.concatenate(
        [jnp.zeros((1, C + D), jnp.float32), beta,
         jnp.zeros((1, pad), jnp.float32)], axis=1)
    bottom = jnp.concatenate(
        [botrow, jnp.zeros((D - 1, C + D + K + pad), jnp.float32)], axis=0)
    At_ref[...] = jnp.concatenate([top, bottom], axis=0)


def _maha_body(X_ref, At_ref, m_ref, s_ref):
    j = pl.program_id(1)
    nj = pl.num_programs(1)
    Xb = X_ref[...]
    BN, D = Xb.shape
    KD = At_ref.shape[1] - 2 * D
    K = KD // D
    ones_lane = (jax.lax.broadcasted_iota(jnp.int32, (BN, D), 1) == 0)
    Xaug = jnp.concatenate(
        [Xb, jnp.where(ones_lane, 1.0, 0.0)], axis=1)
    TtF = jax.lax.dot_general(At_ref[...], Xaug, (((0,), (1,)), ((), ())),
                              preferred_element_type=jnp.float32)
    Tq = TtF[0:KD].reshape(K, D, BN)
    Xt3 = TtF[KD:KD + D].reshape(1, D, BN)
    lb = TtF[KD + D:KD + D + K]
    qT = jnp.sum(Tq * Xt3, axis=1)
    logp = lb + qT
    mk = jnp.max(logp, axis=0, keepdims=True)
    ss = jnp.sum(jnp.exp(logp - mk), axis=0, keepdims=True)

    @pl.when(j == 0)
    def _():
        m_ref[...] = mk.reshape(1, 1, BN)
        s_ref[...] = ss.reshape(1, 1, BN)

    @pl.when(j > 0)
    def _():
        mp = m_ref[...].reshape(1, BN)
        sp = s_ref[...].reshape(1, BN)
        mn = jnp.maximum(mp, mk)
        s_ref[...] = (sp * jnp.exp(mp - mn) + ss * jnp.exp(mk - mn)).reshape(1, 1, BN)
        m_ref[...] = mn.reshape(1, 1, BN)

    @pl.when(j == nj - 1)
    def _():
        mv = m_ref[...].reshape(1, BN)
        sv = s_ref[...].reshape(1, BN)
        mtot = jnp.max(mv)
        stot = jnp.sum(sv * jnp.exp(mv - mtot))
        m_ref[...] = jnp.full((1, 1, BN), mtot, jnp.float32)
        s_ref[...] = jnp.full((1, 1, BN), stot, jnp.float32)


def kernel(X, mu, L, weights, it):
    N, D = X.shape
    K = mu.shape[0]
    CF = K * D + 2 * D
    w2 = weights.reshape(1, K)
    At = pl.pallas_call(
        _prep_body,
        out_shape=jax.ShapeDtypeStruct((2 * D, CF), jnp.float32),
    )(L, mu, w2)

    BN = 512
    PAR = 8
    SEQ = N // (PAR * BN)
    m, s = pl.pallas_call(
        _maha_body,
        grid=(PAR, SEQ),
        in_specs=[pl.BlockSpec((BN, D), lambda i, j: (i * SEQ + j, 0)),
                  pl.BlockSpec((2 * D, CF), lambda i, j: (0, 0))],
        out_specs=[pl.BlockSpec((1, 1, BN), lambda i, j: (i, 0, 0)),
                   pl.BlockSpec((1, 1, BN), lambda i, j: (i, 0, 0))],
        out_shape=[jax.ShapeDtypeStruct((PAR, 1, BN), jnp.float32),
                   jax.ShapeDtypeStruct((PAR, 1, BN), jnp.float32)],
        compiler_params=pltpu.CompilerParams(
            dimension_semantics=("parallel", "arbitrary")),
    )(X, At)

    mv = m[:, 0, 0]
    sv = s[:, 0, 0]
    Mx = jnp.max(mv)
    return -(Mx + jnp.log(jnp.sum(sv * jnp.exp(mv - Mx))))
